# baseline (speedup 1.0000x reference)
import functools
import jax
import jax.numpy as jnp
from jax import lax
from jax.experimental import pallas as pl
from jax.experimental.pallas import tpu as pltpu
from jax.experimental.pallas import tpu_sc as plsc

VOCAB = 100000
EMBED = 64
BATCH = 1024

_info = plsc.get_sparse_core_info()
_NC = _info.num_cores
_NS = _info.num_subcores
_NW = _NC * _NS
_BPW = BATCH // _NW


def _sc_gather(table, idx):
    mesh = plsc.VectorSubcoreMesh(core_axis_name="c", subcore_axis_name="s")

    @functools.partial(
        pl.kernel,
        mesh=mesh,
        out_type=jax.ShapeDtypeStruct((BATCH, EMBED), jnp.float32),
        scratch_types=[
            pltpu.VMEM((_BPW,), jnp.int32),
            pltpu.VMEM((_BPW, EMBED), jnp.float32),
            pltpu.SemaphoreType.DMA,
        ],
        compiler_params=pltpu.CompilerParams(use_tc_tiling_on_sc=False),
    )
    def gather_kernel(table_hbm, idx_hbm, out_hbm, idx_v, rows_v, sem):
        wid = lax.axis_index("s") * _NC + lax.axis_index("c")
        base = wid * _BPW
        pltpu.sync_copy(idx_hbm.at[pl.ds(base, _BPW)], idx_v)
        pltpu.async_copy(table_hbm.at[idx_v], rows_v, sem).wait()
        pltpu.sync_copy(rows_v, out_hbm.at[pl.ds(base, _BPW)])

    return gather_kernel(table, idx)


_TILE_V = 2048
_NT = (VOCAB + _TILE_V - 1) // _TILE_V
_LAST_W = VOCAB - (_NT - 1) * _TILE_V
_NBUF = 4


def _tc_project(emb, W, b2d):
    def mm_kernel(emb_ref, w_ref, b_ref, out_ref, obuf, tailbuf, sems, tail_sem):
        j = pl.program_id(0)
        slot = lax.rem(j, _NBUF)

        @pl.when(j >= _NBUF)
        def _():
            pltpu.make_async_copy(
                obuf.at[slot],
                out_ref.at[:, pl.ds((j - _NBUF) * _TILE_V, _TILE_V)],
                sems.at[slot],
            ).wait()

        val = (
            jnp.dot(emb_ref[...], w_ref[...], preferred_element_type=jnp.float32)
            + b_ref[...]
        )

        @pl.when(j < _NT - 1)
        def _():
            obuf[slot] = val
            pltpu.make_async_copy(
                obuf.at[slot],
                out_ref.at[:, pl.ds(j * _TILE_V, _TILE_V)],
                sems.at[slot],
            ).start()

        @pl.when(j == _NT - 1)
        def _():
            tailbuf[...] = val[:, :_LAST_W]
            pltpu.make_async_copy(
                tailbuf,
                out_ref.at[:, pl.ds((_NT - 1) * _TILE_V, _LAST_W)],
                tail_sem,
            ).start()
            for t in range(_NT - _NBUF, _NT - 1):
                s = t % _NBUF
                pltpu.make_async_copy(
                    obuf.at[s],
                    out_ref.at[:, pl.ds(t * _TILE_V, _TILE_V)],
                    sems.at[s],
                ).wait()
            pltpu.make_async_copy(
                tailbuf,
                out_ref.at[:, pl.ds((_NT - 1) * _TILE_V, _LAST_W)],
                tail_sem,
            ).wait()

    return pl.pallas_call(
        mm_kernel,
        grid=(_NT,),
        in_specs=[
            pl.BlockSpec((BATCH, EMBED), lambda j: (0, 0)),
            pl.BlockSpec((EMBED, _TILE_V), lambda j: (0, j)),
            pl.BlockSpec((1, _TILE_V), lambda j: (0, j)),
        ],
        out_specs=pl.BlockSpec(memory_space=pl.ANY),
        out_shape=jax.ShapeDtypeStruct((BATCH, VOCAB), jnp.float32),
        scratch_shapes=[
            pltpu.VMEM((_NBUF, BATCH, _TILE_V), jnp.float32),
            pltpu.VMEM((BATCH, _LAST_W), jnp.float32),
            pltpu.SemaphoreType.DMA((_NBUF,)),
            pltpu.SemaphoreType.DMA,
        ],
    )(emb, W, b2d)


def kernel(input, table, W, b):
    idx = input.astype(jnp.int32)
    emb = _sc_gather(table, idx)
    return _tc_project(emb, W, b.reshape(1, VOCAB))
